# Initial kernel scaffold; baseline (speedup 1.0000x reference)
import jax
import jax.numpy as jnp
from jax.experimental import pallas as pl


def kernel(x, gamma, beta, W1, b1, W2, b2):
    raise NotImplementedError("write your pallas kernel here")


# speedup vs baseline: 5.7375x; 5.7375x over previous
import functools

import jax
import jax.numpy as jnp
import numpy as np
from jax.experimental import pallas as pl

_N = 32768
_D = 768
_H = 32
_E = 64
_INV_SQRT2 = float(1.0 / np.sqrt(2.0))


def _router_body(x_ref, gamma_ref, beta_ref, w1_ref, b1_ref, w2_ref, b2_ref,
                 probs_ref, ml_ref):
    x = x_ref[...]
    mu = jnp.mean(x, axis=1, keepdims=True)
    xc = x - mu
    var = jnp.mean(xc * xc, axis=1, keepdims=True)
    h = xc * jax.lax.rsqrt(var + 1e-5)
    h = h * gamma_ref[...] + beta_ref[...]

    h1 = jnp.dot(h, w1_ref[...], preferred_element_type=jnp.float32)
    h1 = h1 + b1_ref[...]
    g = 0.5 * h1 * (1.0 + jax.lax.erf(h1 * _INV_SQRT2))

    logits = jnp.dot(g, w2_ref[...], preferred_element_type=jnp.float32)
    logits = logits + b2_ref[...]

    col = jax.lax.broadcasted_iota(jnp.int32, logits.shape, 1)
    m1 = jnp.max(logits, axis=1, keepdims=True)
    i1 = jnp.min(jnp.where(logits == m1, col, _E), axis=1, keepdims=True)
    without1 = jnp.where(col == i1, -jnp.inf, logits)
    m2 = jnp.max(without1, axis=1, keepdims=True)
    i2 = jnp.min(jnp.where(without1 == m2, col, _E), axis=1, keepdims=True)
    mask = (col == i1) | (col == i2)

    ml_ref[...] = jnp.where(mask, logits, -jnp.inf)
    ex = jnp.where(mask, jnp.exp(logits - m1), 0.0)
    probs_ref[...] = ex / jnp.sum(ex, axis=1, keepdims=True)


@functools.partial(jax.jit, static_argnames=("tile", "interpret"))
def _router(x, gamma, beta, w1, b1, w2, b2, tile=1024, interpret=False):
    n, d = x.shape
    grid = (n // tile,)
    return pl.pallas_call(
        _router_body,
        grid=grid,
        in_specs=[
            pl.BlockSpec((tile, d), lambda i: (i, 0)),
            pl.BlockSpec((d,), lambda i: (0,)),
            pl.BlockSpec((d,), lambda i: (0,)),
            pl.BlockSpec((d, _H), lambda i: (0, 0)),
            pl.BlockSpec((_H,), lambda i: (0,)),
            pl.BlockSpec((_H, _E), lambda i: (0, 0)),
            pl.BlockSpec((_E,), lambda i: (0,)),
        ],
        out_specs=[
            pl.BlockSpec((tile, _E), lambda i: (i, 0)),
            pl.BlockSpec((tile, _E), lambda i: (i, 0)),
        ],
        out_shape=[
            jax.ShapeDtypeStruct((n, _E), jnp.float32),
            jax.ShapeDtypeStruct((n, _E), jnp.float32),
        ],
        interpret=interpret,
    )(x, gamma, beta, w1, b1, w2, b2)


def kernel(x, gamma, beta, W1, b1, W2, b2):
    probs, masked_logits = _router(x, gamma, beta, W1, b1, W2, b2)
    return (probs, masked_logits)
